# Initial kernel scaffold; baseline (speedup 1.0000x reference)
import jax
import jax.numpy as jnp
from jax.experimental import pallas as pl


def kernel(x, edge_attr, edge_index, W0, b0, Wm0, bm0, W1, b1, Wm1, bm1, Weu0, beu0, Weu1, beu1):
    raise NotImplementedError("write your pallas kernel here")


# speedup vs baseline: 2.6186x; 2.6186x over previous
import functools

import jax
import jax.numpy as jnp
from jax import lax
from jax.experimental import pallas as pl
from jax.experimental.pallas import tpu as pltpu
from jax.experimental.pallas import tpu_sc as plsc

NN = 10000
NE = 160000
D = 256
DE = 16
HALF = D // 2
NC, NS = 2, 16
RPT = 624

CH_L = 80
NCH_L = (NE // NS) // CH_L
CH_E = 40
NCH_E = (NE // (NC * NS)) // CH_E

_MESH = plsc.VectorSubcoreMesh(
    core_axis_name="c", subcore_axis_name="s", num_cores=NC, num_subcores=NS)


def _copy_tile_rows(src, dst, s, src_base=0, dst_base=0):
  start = s * RPT
  pltpu.sync_copy(src.at[pl.ds(src_base + start, RPT)],
                  dst.at[pl.ds(dst_base + start, RPT)])

  @pl.when(s == NS - 1)
  def _():
    tail = NS * RPT
    pltpu.sync_copy(src.at[pl.ds(src_base + tail, NN - tail)],
                    dst.at[pl.ds(dst_base + tail, NN - tail)])


def _mm_body(a_ref, b_ref, o_ref):
  o_ref[...] = jnp.dot(a_ref[...], b_ref[...],
                       preferred_element_type=jnp.float32)


def _mm_bias_body(a_ref, b_ref, bias_ref, o_ref):
  o_ref[...] = jnp.dot(a_ref[...], b_ref[...],
                       preferred_element_type=jnp.float32) + bias_ref[...]


def _mm_bias_split_body(a_ref, b_ref, bias_ref, o_ref):
  j = pl.program_id(1)
  o_ref[...] = jnp.dot(a_ref[...], b_ref[...],
                       preferred_element_type=jnp.float32) + bias_ref[
                           pl.ds(j, 1), :]


def _matmul(a, b, bias=None, block_m=1000):
  m, k = a.shape
  nout = b.shape[1]
  gm = m // block_m
  in_specs = [
      pl.BlockSpec((block_m, k), lambda i: (i, 0)),
      pl.BlockSpec((k, nout), lambda i: (0, 0)),
  ]
  args = [a, b]
  body = _mm_body
  if bias is not None:
    in_specs.append(pl.BlockSpec((1, nout), lambda i: (0, 0)))
    args.append(bias.reshape(1, nout))
    body = _mm_bias_body
  return pl.pallas_call(
      body,
      grid=(gm,),
      in_specs=in_specs,
      out_specs=pl.BlockSpec((block_m, nout), lambda i: (i, 0)),
      out_shape=jax.ShapeDtypeStruct((m, nout), jnp.float32),
  )(*args)


def _matmul_split(a, b, bias=None, block_m=1000):
  m, k = a.shape
  gm = m // block_m
  in_specs = [
      pl.BlockSpec((block_m, k), lambda i, j: (i, 0)),
      pl.BlockSpec((k, HALF), lambda i, j: (0, j)),
  ]
  args = [a, b]
  body = _mm_body
  if bias is not None:
    in_specs.append(pl.BlockSpec((2, HALF), lambda i, j: (0, 0)))
    args.append(bias.reshape(2, HALF))
    body = _mm_bias_split_body
  return pl.pallas_call(
      body,
      grid=(gm, 2),
      in_specs=in_specs,
      out_specs=pl.BlockSpec((block_m, HALF), lambda i, j: (j * gm + i, 0)),
      out_shape=jax.ShapeDtypeStruct((2 * m, HALF), jnp.float32),
  )(*args)


@functools.partial(
    pl.kernel,
    out_type=jax.ShapeDtypeStruct((NC * NN, DE), jnp.float32),
    mesh=_MESH,
    scratch_types=[
        pltpu.VMEM((CH_E,), jnp.int32),
        pltpu.VMEM((CH_E, DE), jnp.float32),
        pltpu.VMEM_SHARED((NN, DE), jnp.float32),
    ],
    compiler_params=pltpu.CompilerParams(use_tc_tiling_on_sc=False),
)
def _deg_kernel(row_hbm, zeros_hbm, out_hbm, ri, ones_v, acc):
  c = lax.axis_index("c")
  s = lax.axis_index("s")
  wid = c * NS + s
  for e in range(CH_E):
    ones_v[e] = jnp.full((DE,), 1.0, jnp.float32)
  _copy_tile_rows(zeros_hbm, acc, s)
  plsc.subcore_barrier()

  def chunk(j, carry):
    base = wid * (NE // (NC * NS)) + j * CH_E
    pltpu.sync_copy(row_hbm.at[pl.ds(base, CH_E)], ri)
    pltpu.sync_copy(ones_v, acc.at[ri], add=True)
    return carry

  lax.fori_loop(0, NCH_E, chunk, 0)
  plsc.subcore_barrier()
  _copy_tile_rows(acc, out_hbm, s, dst_base=c * NN)


@functools.partial(
    pl.kernel,
    out_type=jax.ShapeDtypeStruct((NC * NN, HALF), jnp.float32),
    mesh=_MESH,
    scratch_types=[
        pltpu.VMEM((NN,), jnp.float32),
        pltpu.VMEM((CH_L,), jnp.int32),
        pltpu.VMEM((CH_L,), jnp.int32),
        pltpu.VMEM((CH_L,), jnp.int32),
        pltpu.VMEM((CH_L + 16,), jnp.float32),
        pltpu.VMEM((CH_L, HALF), jnp.float32),
        pltpu.VMEM((CH_L, HALF), jnp.float32),
        pltpu.VMEM((CH_L, HALF), jnp.float32),
        pltpu.VMEM_SHARED((NN, HALF), jnp.float32),
        pltpu.SemaphoreType.DMA,
    ],
    compiler_params=pltpu.CompilerParams(needs_layout_passes=False),
)
def _layer_kernel(hm_hbm, eap_hbm, row_hbm, col_hbm, dinv_hbm, zeros_hbm,
                  out_hbm, dinv_v, ri_raw, ri_off, ci, norm_v, hrows, eapv,
                  mv, acc, sem):
  c = lax.axis_index("c")
  s = lax.axis_index("s")
  pltpu.sync_copy(dinv_hbm, dinv_v)
  _copy_tile_rows(zeros_hbm, acc, s)
  plsc.subcore_barrier()

  def chunk(j, carry):
    base = s * (NE // NS) + j * CH_L
    pltpu.sync_copy(row_hbm.at[pl.ds(base, CH_L)], ri_raw)
    pltpu.sync_copy(col_hbm.at[pl.ds(base, CH_L)], ci)
    for k in range(CH_L // 16):
      sl = pl.ds(k * 16, 16)
      r16 = ri_raw[sl]
      ri_off[sl] = r16 + c * NN
      norm_v[sl] = plsc.load_gather(dinv_v, [r16]) * plsc.load_gather(
          dinv_v, [ci[sl]])
    pltpu.async_copy(hm_hbm.at[ri_off], hrows, sem).wait()
    pltpu.sync_copy(eap_hbm.at[pl.ds(c * NE + base, CH_L)], eapv)

    def edge(e, inner):
      nrm = norm_v[pl.ds(e, 16)][0]
      for l in range(HALF // 16):
        sl = pl.ds(l * 16, 16)
        mv[e, sl] = jnp.maximum(hrows[e, sl] + eapv[e, sl], 0.0) * nrm
      return inner

    lax.fori_loop(0, CH_L, edge, 0)
    pltpu.sync_copy(mv, acc.at[ci], add=True)
    return carry

  lax.fori_loop(0, NCH_L, chunk, 0)
  plsc.subcore_barrier()
  _copy_tile_rows(acc, out_hbm, s, dst_base=c * NN)


@functools.partial(
    pl.kernel,
    out_type=jax.ShapeDtypeStruct((NE, DE), jnp.float32),
    mesh=_MESH,
    scratch_types=[
        pltpu.VMEM((CH_E,), jnp.int32),
        pltpu.VMEM((CH_E,), jnp.int32),
        pltpu.VMEM((CH_E, DE), jnp.float32),
        pltpu.VMEM((CH_E, DE), jnp.float32),
        pltpu.VMEM((CH_E, DE), jnp.float32),
        pltpu.VMEM((CH_E, DE), jnp.float32),
        pltpu.SemaphoreType.DMA,
    ],
    compiler_params=pltpu.CompilerParams(use_tc_tiling_on_sc=False),
)
def _edge_update_kernel(g_hbm, eaw_hbm, row_hbm, col_hbm, out_hbm,
                        ri, ci, gr, gc, ew, ov, sem):
  c = lax.axis_index("c")
  s = lax.axis_index("s")
  wid = c * NS + s

  def chunk(j, carry):
    base = wid * (NE // (NC * NS)) + j * CH_E
    pltpu.sync_copy(row_hbm.at[pl.ds(base, CH_E)], ri)
    pltpu.sync_copy(col_hbm.at[pl.ds(base, CH_E)], ci)
    pltpu.async_copy(g_hbm.at[ri], gr, sem).wait()
    pltpu.async_copy(g_hbm.at[ci], gc, sem).wait()
    pltpu.sync_copy(eaw_hbm.at[pl.ds(base, CH_E)], ew)
    for e in range(CH_E):
      ov[e] = jnp.maximum(gr[e] + gc[e] + ew[e], 0.0)
    pltpu.sync_copy(ov, out_hbm.at[pl.ds(base, CH_E)])
    return carry

  lax.fori_loop(0, NCH_E, chunk, 0)


def kernel(x, edge_attr, edge_index, W0, b0, Wm0, bm0, W1, b1, Wm1, bm1,
           Weu0, beu0, Weu1, beu1):
  row = edge_index[0]
  col = edge_index[1]
  zeros_l = jnp.zeros((NN, HALF), jnp.float32)
  zeros_e = jnp.zeros((NN, DE), jnp.float32)

  degout = _deg_kernel(row, zeros_e)
  deg = degout[:NN, 0] + degout[NN:, 0]
  dinv = jnp.where(deg > 0, lax.rsqrt(jnp.maximum(deg, 1e-12)), 0.0)

  def conv(xin, ea, W, b, Wm, bm):
    Wc = _matmul(W, Wm[:D], block_m=D)
    hm = _matmul_split(xin, Wc, block_m=1000)
    eap = _matmul_split(ea, Wm[D:], bias=bm, block_m=2000)
    acc = _layer_kernel(hm, eap, row, col, dinv, zeros_l)
    return jnp.concatenate([acc[:NN], acc[NN:]], axis=1) + b

  def edge_update(xin, ea, Weu, beu):
    g = _matmul(xin, 0.5 * Weu[:D], block_m=1000)
    eaw = _matmul(ea, Weu[D:], bias=beu, block_m=2000)
    return _edge_update_kernel(g, eaw, row, col)

  x1 = conv(x, edge_attr, W0, b0, Wm0, bm0)
  ea1 = edge_update(x1, edge_attr, Weu0, beu0)
  x2 = conv(x1, ea1, W1, b1, Wm1, bm1)
  ea2 = edge_update(x2, ea1, Weu1, beu1)
  return (x2, ea2)
